# baseline (speedup 1.0000x reference)
import functools

import jax
import jax.numpy as jnp
from jax import lax
from jax.experimental import pallas as pl
from jax.experimental.pallas import tpu as pltpu
from jax.experimental.pallas import tpu_sc as plsc

BATCH = 16384
HIDDEN = 64
FEAT_DIM = 20

_NC = 2
_NS = 16
_NW = _NC * _NS
_BPW = BATCH // _NW


def _sc_gather_body(uid_hbm, mid_hbm, utab_hbm, mtab_hbm,
                    uout_hbm, mout_hbm,
                    uidx_v, midx_v, sem_u, sem_m):
    wid = lax.axis_index("s") * _NC + lax.axis_index("c")
    base = wid * _BPW
    pltpu.sync_copy(uid_hbm.at[pl.ds(base, _BPW)], uidx_v)
    pltpu.sync_copy(mid_hbm.at[pl.ds(base, _BPW)], midx_v)

    def fire(g, carry):
        uvec = uidx_v[pl.ds(g * 16, 16)]
        mvec = midx_v[pl.ds(g * 16, 16)]
        for j in range(16):
            i = g * 16 + j
            pltpu.make_async_copy(
                utab_hbm.at[pl.ds(uvec[j], 1)],
                uout_hbm.at[pl.ds(base + i, 1)], sem_u,
            ).start()
            pltpu.make_async_copy(
                mtab_hbm.at[pl.ds(mvec[j], 1)],
                mout_hbm.at[pl.ds(base + i, 1)], sem_m,
            ).start()
        return carry

    lax.fori_loop(0, _BPW // 16, fire, 0)
    pltpu.make_async_copy(
        utab_hbm.at[pl.ds(0, _BPW)], uout_hbm.at[pl.ds(base, _BPW)], sem_u
    ).wait()
    pltpu.make_async_copy(
        mtab_hbm.at[pl.ds(0, _BPW)], mout_hbm.at[pl.ds(base, _BPW)], sem_m
    ).wait()


@functools.cache
def _sc_gather():
    return pl.kernel(
        _sc_gather_body,
        out_type=(
            jax.ShapeDtypeStruct((BATCH, HIDDEN), jnp.float32),
            jax.ShapeDtypeStruct((BATCH, HIDDEN), jnp.float32),
        ),
        mesh=plsc.VectorSubcoreMesh(core_axis_name="c", subcore_axis_name="s"),
        scratch_types=[
            pltpu.VMEM((_BPW,), jnp.int32),
            pltpu.VMEM((_BPW,), jnp.int32),
            pltpu.SemaphoreType.DMA,
            pltpu.SemaphoreType.DMA,
        ],
    )


def _tc_combine_body(feat_ref, u_ref, m_ref, w_ref, b_ref, out_ref):
    proj = jnp.dot(feat_ref[...], w_ref[...],
                   preferred_element_type=jnp.float32) + b_ref[...]
    out_ref[...] = jnp.sum(u_ref[...] * (m_ref[...] + proj),
                           axis=1).reshape(out_ref.shape)


_TC_ROWS = 2048


def _tc_combine(movie_features, user_emb, movie_emb, W, b2d):
    grid = (BATCH // _TC_ROWS,)
    out = pl.pallas_call(
        _tc_combine_body,
        grid=grid,
        in_specs=[
            pl.BlockSpec((_TC_ROWS, FEAT_DIM), lambda i: (i, 0)),
            pl.BlockSpec((_TC_ROWS, HIDDEN), lambda i: (i, 0)),
            pl.BlockSpec((_TC_ROWS, HIDDEN), lambda i: (i, 0)),
            pl.BlockSpec((FEAT_DIM, HIDDEN), lambda i: (0, 0)),
            pl.BlockSpec((1, HIDDEN), lambda i: (0, 0)),
        ],
        out_specs=pl.BlockSpec((_TC_ROWS,), lambda i: (i,)),
        out_shape=jax.ShapeDtypeStruct((BATCH,), jnp.float32),
    )(movie_features, user_emb, movie_emb, W, b2d)
    return out


@jax.jit
def kernel(user_ids, movie_ids, movie_features, user_table, movie_table, W, b):
    uids = user_ids.astype(jnp.int32)
    mids = movie_ids.astype(jnp.int32)
    user_emb, movie_emb = _sc_gather()(uids, mids, user_table, movie_table)
    return _tc_combine(movie_features, user_emb, movie_emb, W,
                       b.reshape(1, HIDDEN))


# speedup vs baseline: 2.1653x; 2.1653x over previous
import functools

import jax
import jax.numpy as jnp
from jax import lax
from jax.experimental import pallas as pl
from jax.experimental.pallas import tpu as pltpu
from jax.experimental.pallas import tpu_sc as plsc

BATCH = 16384
HIDDEN = 64
FEAT_DIM = 20
_NC = 2
_NS = 16
_NW = _NC * _NS
_BPW = BATCH // _NW


_HB = _BPW // 2


def _sc_gather_body(uid_hbm, mid_hbm, utab_hbm, mtab_hbm,
                    uout_hbm, mout_hbm,
                    uid_v, mid_v, rows_v, gsem, wsem):
    wid = lax.axis_index("s") * _NC + lax.axis_index("c")
    base = wid * _BPW
    pltpu.sync_copy(uid_hbm.at[pl.ds(base, _BPW)], uid_v)
    pltpu.sync_copy(mid_hbm.at[pl.ds(base, _BPW)], mid_v)

    def phase(ids_ref, tab_hbm, out_hbm, half, par, first):
        buf = rows_v.at[par]

        def fire(g, carry):
            vec = ids_ref[pl.ds(half * _HB + g * 16, 16)]
            for j in range(16):
                pltpu.make_async_copy(
                    tab_hbm.at[pl.ds(vec[j], 1)],
                    buf.at[pl.ds(g * 16 + j, 1)],
                    gsem,
                ).start()
            return carry

        if not first:
            pltpu.make_async_copy(
                out_hbm.at[pl.ds(0, _HB)], buf, wsem).wait()
        lax.fori_loop(0, _HB // 16, fire, 0)
        pltpu.make_async_copy(tab_hbm.at[pl.ds(0, _HB)], buf, gsem).wait()
        pltpu.make_async_copy(
            buf, out_hbm.at[pl.ds(base + half * _HB, _HB)], wsem
        ).start()

    phase(uid_v, utab_hbm, uout_hbm, 0, 0, True)
    phase(uid_v, utab_hbm, uout_hbm, 1, 1, True)
    phase(mid_v, mtab_hbm, mout_hbm, 0, 0, False)
    phase(mid_v, mtab_hbm, mout_hbm, 1, 1, False)
    pltpu.make_async_copy(mout_hbm.at[pl.ds(0, _HB)], rows_v.at[0], wsem).wait()
    pltpu.make_async_copy(mout_hbm.at[pl.ds(0, _HB)], rows_v.at[1], wsem).wait()


@functools.cache
def _sc_gather():
    return pl.kernel(
        _sc_gather_body,
        out_type=(
            jax.ShapeDtypeStruct((BATCH, HIDDEN), jnp.float32),
            jax.ShapeDtypeStruct((BATCH, HIDDEN), jnp.float32),
        ),
        mesh=plsc.VectorSubcoreMesh(core_axis_name="c", subcore_axis_name="s"),
        scratch_types=[
            pltpu.VMEM((_BPW,), jnp.int32),
            pltpu.VMEM((_BPW,), jnp.int32),
            pltpu.VMEM((2, _HB, HIDDEN), jnp.float32),
            pltpu.SemaphoreType.DMA,
            pltpu.SemaphoreType.DMA,
        ],
    )


def _tc_combine_body(feat_ref, u_ref, m_ref, w_ref, b_ref, out_ref):
    proj = jnp.dot(feat_ref[...], w_ref[...],
                   preferred_element_type=jnp.float32) + b_ref[...]
    out_ref[...] = jnp.sum(u_ref[...] * (m_ref[...] + proj),
                           axis=1).reshape(out_ref.shape)


_TC_ROWS = 2048


def _tc_combine(movie_features, user_emb, movie_emb, W, b2d):
    grid = (BATCH // _TC_ROWS,)
    out = pl.pallas_call(
        _tc_combine_body,
        grid=grid,
        in_specs=[
            pl.BlockSpec((_TC_ROWS, FEAT_DIM), lambda i: (i, 0)),
            pl.BlockSpec((_TC_ROWS, HIDDEN), lambda i: (i, 0)),
            pl.BlockSpec((_TC_ROWS, HIDDEN), lambda i: (i, 0)),
            pl.BlockSpec((FEAT_DIM, HIDDEN), lambda i: (0, 0)),
            pl.BlockSpec((1, HIDDEN), lambda i: (0, 0)),
        ],
        out_specs=pl.BlockSpec((_TC_ROWS,), lambda i: (i,)),
        out_shape=jax.ShapeDtypeStruct((BATCH,), jnp.float32),
    )(movie_features, user_emb, movie_emb, W, b2d)
    return out


@jax.jit
def kernel(user_ids, movie_ids, movie_features, user_table, movie_table, W, b):
    uids = user_ids.astype(jnp.int32)
    mids = movie_ids.astype(jnp.int32)
    user_emb, movie_emb = _sc_gather()(uids, mids, user_table, movie_table)
    return _tc_combine(movie_features, user_emb, movie_emb, W,
                       b.reshape(1, HIDDEN))
